# baseline (speedup 1.0000x reference)
import functools

import jax
import jax.numpy as jnp
from jax import lax
from jax.experimental import pallas as pl
from jax.experimental.pallas import tpu as pltpu
from jax.experimental.pallas import tpu_sc as plsc

B = 16384
D_IND = 64
D_ROLE = 128

_info = plsc.get_sparse_core_info()
NC, NS = _info.num_cores, _info.num_subcores
NW = NC * NS
B_PER_W = B // NW
CHUNK = 128
N_CHUNKS = B_PER_W // CHUNK


def _make_kernel():
    mesh = plsc.VectorSubcoreMesh(core_axis_name="c", subcore_axis_name="s")

    @functools.partial(
        pl.kernel,
        mesh=mesh,
        compiler_params=pltpu.CompilerParams(use_tc_tiling_on_sc=False),
        out_type=[
            jax.ShapeDtypeStruct((B, D_ROLE), jnp.float32),
            jax.ShapeDtypeStruct((2 * B, D_IND), jnp.float32),
            jax.ShapeDtypeStruct((2 * B, D_IND), jnp.float32),
        ],
        scratch_types=[
            pltpu.VMEM((6, CHUNK), jnp.int32),
            pltpu.VMEM((CHUNK, D_ROLE), jnp.float32),
            pltpu.VMEM((CHUNK, D_IND), jnp.float32),
            pltpu.VMEM((CHUNK, D_IND), jnp.float32),
            pltpu.VMEM((CHUNK, D_IND), jnp.float32),
            pltpu.SemaphoreType.DMA,
        ],
    )
    def gather_kernel(idx_hbm, ind_hbm, role_hbm, out1, out2, out3,
                      idx_v, role_v, subj_v, obj_v, neg_v, sem):
        wid = lax.axis_index("s") * NC + lax.axis_index("c")
        for ci in range(N_CHUNKS):
            base = wid * B_PER_W + ci * CHUNK
            pltpu.sync_copy(idx_hbm.at[:, pl.ds(base, CHUNK)], idx_v)
            c_subj = pltpu.async_copy(ind_hbm.at[idx_v.at[0]], subj_v, sem)
            c_obj = pltpu.async_copy(ind_hbm.at[idx_v.at[1]], obj_v, sem)
            c_neg = pltpu.async_copy(ind_hbm.at[idx_v.at[2]], neg_v, sem)
            c_role = pltpu.async_copy(role_hbm.at[idx_v.at[3]], role_v, sem)
            c_subj.wait()
            c_obj.wait()
            c_neg.wait()
            c_role.wait()
            w1 = pltpu.async_copy(role_v, out1.at[pl.ds(base, CHUNK)], sem)
            w2 = pltpu.async_copy(subj_v, out2.at[idx_v.at[4]], sem)
            w3 = pltpu.async_copy(obj_v, out2.at[idx_v.at[5]], sem)
            w4 = pltpu.async_copy(subj_v, out3.at[idx_v.at[4]], sem)
            w5 = pltpu.async_copy(neg_v, out3.at[idx_v.at[5]], sem)
            w1.wait()
            w2.wait()
            w3.wait()
            w4.wait()
            w5.wait()

    return gather_kernel


_gather = _make_kernel()


def kernel(data, ind_table, role_table):
    neg_key = jax.random.key(42)
    neg = jax.random.randint(neg_key, (data.shape[0],), 0, ind_table.shape[0],
                             dtype=jnp.int32)
    rows = jnp.arange(data.shape[0], dtype=jnp.int32)
    idx = jnp.stack(
        [data[:, 0], data[:, 2], neg, data[:, 1], 2 * rows, 2 * rows + 1],
        axis=0)
    out1, out2f, out3f = _gather(idx, ind_table, role_table)
    out2 = out2f.reshape(data.shape[0], 2 * D_IND)
    out3 = out3f.reshape(data.shape[0], 2 * D_IND)
    return (out1, out2, out3)


# speedup vs baseline: 1.0055x; 1.0055x over previous
import functools

import jax
import jax.numpy as jnp
from jax import lax
from jax.experimental import pallas as pl
from jax.experimental.pallas import tpu as pltpu
from jax.experimental.pallas import tpu_sc as plsc

B = 16384
D_IND = 64
D_ROLE = 128

_info = plsc.get_sparse_core_info()
NC, NS = _info.num_cores, _info.num_subcores
NW = NC * NS
B_PER_W = B // NW
CHUNK = 128
N_CHUNKS = B_PER_W // CHUNK


def _make_kernel():
    mesh = plsc.VectorSubcoreMesh(core_axis_name="c", subcore_axis_name="s")

    @functools.partial(
        pl.kernel,
        mesh=mesh,
        compiler_params=pltpu.CompilerParams(use_tc_tiling_on_sc=False),
        out_type=[
            jax.ShapeDtypeStruct((B, D_ROLE), jnp.float32),
            jax.ShapeDtypeStruct((2 * B, D_IND), jnp.float32),
            jax.ShapeDtypeStruct((2 * B, D_IND), jnp.float32),
        ],
        scratch_types=[
            pltpu.VMEM((2, 6, CHUNK), jnp.int32),
            pltpu.VMEM((2, CHUNK, D_ROLE), jnp.float32),
            pltpu.VMEM((2, CHUNK, D_IND), jnp.float32),
            pltpu.VMEM((2, CHUNK, D_IND), jnp.float32),
            pltpu.VMEM((2, CHUNK, D_IND), jnp.float32),
            pltpu.SemaphoreType.DMA,
            pltpu.SemaphoreType.DMA,
            pltpu.SemaphoreType.DMA,
            pltpu.SemaphoreType.DMA,
        ],
    )
    def gather_kernel(idx_hbm, ind_hbm, role_hbm, out1, out2, out3,
                      idx_v, role_v, subj_v, obj_v, neg_v,
                      gsem0, gsem1, wsem0, wsem1):
        wid = lax.axis_index("s") * NC + lax.axis_index("c")
        gsems = (gsem0, gsem1)
        wsems = (wsem0, wsem1)
        gathers = {}
        writes = {}

        def fire_gathers(ci):
            b = ci % 2
            base = wid * B_PER_W + ci * CHUNK
            pltpu.sync_copy(idx_hbm.at[:, pl.ds(base, CHUNK)], idx_v.at[b])
            gathers[ci] = [
                pltpu.async_copy(ind_hbm.at[idx_v.at[b, 0]], subj_v.at[b],
                                 gsems[b]),
                pltpu.async_copy(ind_hbm.at[idx_v.at[b, 1]], obj_v.at[b],
                                 gsems[b]),
                pltpu.async_copy(ind_hbm.at[idx_v.at[b, 2]], neg_v.at[b],
                                 gsems[b]),
                pltpu.async_copy(role_hbm.at[idx_v.at[b, 3]], role_v.at[b],
                                 gsems[b]),
            ]

        def fire_writes(ci):
            b = ci % 2
            base = wid * B_PER_W + ci * CHUNK
            for d in gathers.pop(ci):
                d.wait()
            writes[ci] = [
                pltpu.async_copy(role_v.at[b], out1.at[pl.ds(base, CHUNK)],
                                 wsems[b]),
                pltpu.async_copy(subj_v.at[b], out2.at[idx_v.at[b, 4]],
                                 wsems[b]),
                pltpu.async_copy(obj_v.at[b], out2.at[idx_v.at[b, 5]],
                                 wsems[b]),
                pltpu.async_copy(subj_v.at[b], out3.at[idx_v.at[b, 4]],
                                 wsems[b]),
                pltpu.async_copy(neg_v.at[b], out3.at[idx_v.at[b, 5]],
                                 wsems[b]),
            ]

        fire_gathers(0)
        for ci in range(1, N_CHUNKS):
            if ci >= 2:
                for d in writes.pop(ci - 2):
                    d.wait()
            fire_gathers(ci)
            fire_writes(ci - 1)
        fire_writes(N_CHUNKS - 1)
        for ci in list(writes):
            for d in writes.pop(ci):
                d.wait()

    return gather_kernel


_gather = _make_kernel()


def kernel(data, ind_table, role_table):
    neg_key = jax.random.key(42)
    neg = jax.random.randint(neg_key, (data.shape[0],), 0, ind_table.shape[0],
                             dtype=jnp.int32)
    rows = jnp.arange(data.shape[0], dtype=jnp.int32)
    idx = jnp.stack(
        [data[:, 0], data[:, 2], neg, data[:, 1], 2 * rows, 2 * rows + 1],
        axis=0)
    out1, out2f, out3f = _gather(idx, ind_table, role_table)
    out2 = out2f.reshape(data.shape[0], 2 * D_IND)
    out3 = out3f.reshape(data.shape[0], 2 * D_IND)
    return (out1, out2, out3)
